# baseline (speedup 1.0000x reference)
import functools
import jax
import jax.numpy as jnp
from jax import lax
from jax.experimental import pallas as pl
from jax.experimental.pallas import tpu as pltpu
from jax.experimental.pallas import tpu_sc as plsc

DIM = 64
BATCH = 16384
LANES = 16
IDXW = 128

_info = plsc.get_sparse_core_info()
NC = _info.num_cores
NS = _info.num_subcores
NW = NC * NS
BPW = BATCH // NW
NIDX = BPW // IDXW
NGRP = BPW // LANES
TPAD = 17

_mesh = plsc.VectorSubcoreMesh(core_axis_name="c", subcore_axis_name="s")


@functools.partial(
    pl.kernel,
    mesh=_mesh,
    out_type=jax.ShapeDtypeStruct((BATCH,), jnp.float32),
    scratch_types=[
        pltpu.VMEM((NIDX, IDXW), jnp.int32),
        pltpu.VMEM((NIDX, IDXW), jnp.int32),
        pltpu.VMEM((BPW, DIM), jnp.float32),
        pltpu.VMEM((BPW, DIM), jnp.float32),
        pltpu.VMEM((BPW,), jnp.float32),
        pltpu.VMEM((LANES * TPAD,), jnp.float32),
        pltpu.SemaphoreType.DMA,
    ],
    compiler_params=pltpu.CompilerParams(
        needs_layout_passes=False, use_tc_tiling_on_sc=False),
)
def _w2v(cw_hbm, xw_hbm, ct_hbm, xt_hbm, out_hbm,
         ci_v, xi_v, cr_v, xr_v, o_v, tb_v, sem):
    wid = lax.axis_index("s") * NC + lax.axis_index("c")
    base = wid * BPW

    for j in range(NIDX):
        pltpu.sync_copy(cw_hbm.at[pl.ds(base + j * IDXW, IDXW)], ci_v.at[j])
        pltpu.sync_copy(xw_hbm.at[pl.ds(base + j * IDXW, IDXW)], xi_v.at[j])

    copies = []
    for j in range(NIDX):
        copies.append(pltpu.async_copy(
            ct_hbm.at[ci_v.at[j]], cr_v.at[pl.ds(j * IDXW, IDXW)], sem))
        copies.append(pltpu.async_copy(
            xt_hbm.at[xi_v.at[j]], xr_v.at[pl.ds(j * IDXW, IDXW)], sem))
    for c in copies:
        c.wait()

    lane = lax.iota(jnp.int32, LANES)

    def group(g, carry):
        r0 = g * LANES
        for j in range(LANES):
            row = r0 + j
            acc = cr_v[row, pl.ds(0, LANES)] * xr_v[row, pl.ds(0, LANES)]
            for k in range(1, DIM // LANES):
                acc = acc + (cr_v[row, pl.ds(k * LANES, LANES)]
                             * xr_v[row, pl.ds(k * LANES, LANES)])
            tb_v[pl.ds(j * TPAD, LANES)] = acc
        tot = plsc.load_gather(tb_v, [lane * TPAD])
        for i in range(1, LANES):
            tot = tot + plsc.load_gather(tb_v, [lane * TPAD + i])
        o_v[pl.ds(r0, LANES)] = tot
        return carry

    lax.fori_loop(0, NGRP, group, 0)
    pltpu.sync_copy(o_v, out_hbm.at[pl.ds(base, BPW)])


def kernel(center_word, context_word, center_table, context_table):
    return _w2v(center_word.astype(jnp.int32), context_word.astype(jnp.int32),
                center_table, context_table)


# speedup vs baseline: 1.4633x; 1.4633x over previous
import functools
import jax
import jax.numpy as jnp
from jax import lax
from jax.experimental import pallas as pl
from jax.experimental.pallas import tpu as pltpu
from jax.experimental.pallas import tpu_sc as plsc

DIM = 64
BATCH = 16384
LANES = 16
CHUNK = 16

_info = plsc.get_sparse_core_info()
NC = _info.num_cores
NS = _info.num_subcores
NW = NC * NS
BPW = BATCH // NW
NCHUNK = BPW // CHUNK
TPAD = 17

_mesh = plsc.VectorSubcoreMesh(core_axis_name="c", subcore_axis_name="s")


@functools.partial(
    pl.kernel,
    mesh=_mesh,
    out_type=jax.ShapeDtypeStruct((BATCH,), jnp.float32),
    scratch_types=[
        pltpu.VMEM((BPW,), jnp.int32),
        pltpu.VMEM((BPW,), jnp.int32),
        pltpu.VMEM((CHUNK, 8, DIM), jnp.float32),
        pltpu.VMEM((CHUNK, 8, DIM), jnp.float32),
        pltpu.VMEM((BPW,), jnp.float32),
        pltpu.VMEM((LANES * TPAD,), jnp.float32),
        pltpu.SemaphoreType.DMA,
    ],
    compiler_params=pltpu.CompilerParams(needs_layout_passes=False),
)
def _w2v(cw_hbm, xw_hbm, ct_hbm, xt_hbm, out_hbm,
         ci_v, xi_v, cs_v, xs_v, o_v, tb_v, sem):
    wid = lax.axis_index("s") * NC + lax.axis_index("c")
    base = wid * BPW

    pltpu.sync_copy(cw_hbm.at[pl.ds(base, BPW)], ci_v)
    pltpu.sync_copy(xw_hbm.at[pl.ds(base, BPW)], xi_v)

    lane = lax.iota(jnp.int32, LANES)

    def chunk_body(g, carry):
        r0 = g * CHUNK
        icv = ci_v[pl.ds(r0, CHUNK)]
        ixv = xi_v[pl.ds(r0, CHUNK)]
        ict = (icv >> 3) << 3
        ixt = (ixv >> 3) << 3
        icq = icv & 7
        ixq = ixv & 7
        for u in range(CHUNK):
            tc = pl.multiple_of(ict[u], 8)
            tx = pl.multiple_of(ixt[u], 8)
            pltpu.async_copy(ct_hbm.at[pl.ds(tc, 8), :], cs_v.at[u], sem)
            pltpu.async_copy(xt_hbm.at[pl.ds(tx, 8), :], xs_v.at[u], sem)
        for _ in range(2 * CHUNK):
            pltpu.make_async_copy(ct_hbm.at[pl.ds(0, 8), :], cs_v.at[0], sem).wait()
        for u in range(CHUNK):
            qc = icq[u]
            qx = ixq[u]
            acc = cs_v[u, qc, pl.ds(0, LANES)] * xs_v[u, qx, pl.ds(0, LANES)]
            for k in range(1, DIM // LANES):
                acc = acc + (cs_v[u, qc, pl.ds(k * LANES, LANES)]
                             * xs_v[u, qx, pl.ds(k * LANES, LANES)])
            tb_v[pl.ds(u * TPAD, LANES)] = acc
        tot = plsc.load_gather(tb_v, [lane * TPAD])
        for i in range(1, LANES):
            tot = tot + plsc.load_gather(tb_v, [lane * TPAD + i])
        o_v[pl.ds(r0, LANES)] = tot
        return carry

    lax.fori_loop(0, NCHUNK, chunk_body, 0)
    pltpu.sync_copy(o_v, out_hbm.at[pl.ds(base, BPW)])


def kernel(center_word, context_word, center_table, context_table):
    return _w2v(center_word.astype(jnp.int32), context_word.astype(jnp.int32),
                center_table, context_table)


# speedup vs baseline: 2.3416x; 1.6003x over previous
import functools
import jax
import jax.numpy as jnp
from jax import lax
from jax.experimental import pallas as pl
from jax.experimental.pallas import tpu as pltpu
from jax.experimental.pallas import tpu_sc as plsc

DIM = 64
BATCH = 16384
LANES = 16
CHUNK = 2
NCPG = LANES // CHUNK

_info = plsc.get_sparse_core_info()
NC = _info.num_cores
NS = _info.num_subcores
NW = NC * NS
BPW = BATCH // NW
NGRP = BPW // LANES
TPAD = 17

_mesh = plsc.VectorSubcoreMesh(core_axis_name="c", subcore_axis_name="s")


@functools.partial(
    pl.kernel,
    mesh=_mesh,
    out_type=jax.ShapeDtypeStruct((BATCH,), jnp.float32),
    scratch_types=[
        pltpu.VMEM((BPW,), jnp.int32),
        pltpu.VMEM((BPW,), jnp.int32),
        pltpu.VMEM((2, CHUNK, DIM, 128), jnp.float32),
        pltpu.VMEM((2, CHUNK, DIM, 128), jnp.float32),
        pltpu.VMEM((BPW,), jnp.float32),
        pltpu.VMEM((LANES * TPAD,), jnp.float32),
        pltpu.SemaphoreType.DMA,
        pltpu.SemaphoreType.DMA,
    ],
    compiler_params=pltpu.CompilerParams(needs_layout_passes=False),
)
def _w2v(cw_hbm, xw_hbm, ctT_hbm, xtT_hbm, out_hbm,
         ci_v, xi_v, cs_v, xs_v, o_v, tb_v, sem0, sem1):
    wid = lax.axis_index("s") * NC + lax.axis_index("c")
    base = wid * BPW

    pltpu.sync_copy(cw_hbm.at[pl.ds(base, BPW)], ci_v)
    pltpu.sync_copy(xw_hbm.at[pl.ds(base, BPW)], xi_v)

    lane = lax.iota(jnp.int32, LANES)
    sems = (sem0, sem1)

    def fire(icv, ixv, u0, buf):
        sem = sems[buf]
        for u in range(CHUNK):
            bc = pl.multiple_of((icv[u0 + u] >> 7) << 7, 128)
            bx = pl.multiple_of((ixv[u0 + u] >> 7) << 7, 128)
            pltpu.async_copy(ctT_hbm.at[:, pl.ds(bc, 128)], cs_v.at[buf, u], sem)
            pltpu.async_copy(xtT_hbm.at[:, pl.ds(bx, 128)], xs_v.at[buf, u], sem)

    def drain(buf):
        for _ in range(2 * CHUNK):
            pltpu.make_async_copy(
                ctT_hbm.at[:, pl.ds(0, 128)], cs_v.at[0, 0], sems[buf]).wait()

    def compute(icv, ixv, u0, buf):
        for u in range(CHUNK):
            qc = jnp.full((LANES,), icv[u0 + u] & 127, jnp.int32)
            qx = jnp.full((LANES,), ixv[u0 + u] & 127, jnp.int32)
            acc = None
            for k in range(DIM // LANES):
                rows = lane + (k * LANES)
                cv = plsc.load_gather(cs_v.at[buf, u], [rows, qc])
                xv = plsc.load_gather(xs_v.at[buf, u], [rows, qx])
                p = cv * xv
                acc = p if acc is None else acc + p
            tb_v[pl.ds((u0 + u) * TPAD, LANES)] = acc

    def group_body(g, carry):
        r0 = g * LANES
        icv = ci_v[pl.ds(r0, LANES)]
        ixv = xi_v[pl.ds(r0, LANES)]
        fire(icv, ixv, 0, 0)
        for c in range(NCPG):
            if c + 1 < NCPG:
                fire(icv, ixv, (c + 1) * CHUNK, (c + 1) % 2)
            drain(c % 2)
            compute(icv, ixv, c * CHUNK, c % 2)
        tot = plsc.load_gather(tb_v, [lane * TPAD])
        for i in range(1, LANES):
            tot = tot + plsc.load_gather(tb_v, [lane * TPAD + i])
        o_v[pl.ds(r0, LANES)] = tot
        return carry

    lax.fori_loop(0, NGRP, group_body, 0)
    pltpu.sync_copy(o_v, out_hbm.at[pl.ds(base, BPW)])


def kernel(center_word, context_word, center_table, context_table):
    return _w2v(center_word.astype(jnp.int32), context_word.astype(jnp.int32),
                jnp.transpose(center_table), jnp.transpose(context_table))


# speedup vs baseline: 3.8721x; 1.6536x over previous
import functools
import jax
import jax.numpy as jnp
from jax import lax
from jax.experimental import pallas as pl
from jax.experimental.pallas import tpu as pltpu
from jax.experimental.pallas import tpu_sc as plsc

DIM = 64
BATCH = 16384
LANES = 16
NBLK = 7813
CPB = 4
CHW = CPB * 128
NCHK = 62
LCAP = 2048
RING = 8

_info = plsc.get_sparse_core_info()
NC = _info.num_cores
NS = _info.num_subcores
NW = NC * NS
BPW = BATCH // NW
NGRP = BPW // LANES
TPAD = 17

_mesh = plsc.VectorSubcoreMesh(core_axis_name="c", subcore_axis_name="s")
_params = pltpu.CompilerParams(needs_layout_passes=False)

_DNUMS = lax.GatherDimensionNumbers(
    offset_dims=(), collapsed_slice_dims=(0,), start_index_map=(0,))


def _dyn_gather(v, j):
    return lax.gather(v, j[:, None], _DNUMS, slice_sizes=(1,),
                      mode=lax.GatherScatterMode.PROMISE_IN_BOUNDS)


@functools.partial(
    pl.kernel,
    mesh=_mesh,
    out_type=jax.ShapeDtypeStruct((BATCH * DIM,), jnp.float32),
    scratch_types=[
        pltpu.VMEM((BATCH,), jnp.int32),
        pltpu.VMEM((LCAP + LANES,), jnp.int32),
        pltpu.VMEM((LCAP + LANES,), jnp.int32),
        pltpu.VMEM((2, DIM, CHW), jnp.float32),
        pltpu.VMEM((RING * DIM,), jnp.float32),
        pltpu.SemaphoreType.DMA,
        pltpu.SemaphoreType.DMA,
    ],
    compiler_params=_params,
)
def _scan_gather(iw_hbm, tT_hbm, rows_hbm,
                 idx_v, bl_v, il_v, st_v, rb_v, sem_in, sem_out):
    wid = lax.axis_index("s") * NC + lax.axis_index("c")
    bs = wid * 244 + jnp.minimum(wid, 5)
    bn = 244 + (wid < 5).astype(jnp.int32)
    be = bs + bn

    pltpu.sync_copy(iw_hbm, idx_v)
    lane = lax.iota(jnp.int32, LANES)

    def prescan(g, cnt):
        v = idx_v[pl.ds(g * LANES, LANES)]
        blk = v >> 7
        m = (blk >= bs) & (blk < be)
        bl = lane + (g * LANES)
        plsc.store_compressed(bl_v.at[pl.ds(cnt, LANES)], bl, mask=m)
        plsc.store_compressed(il_v.at[pl.ds(cnt, LANES)], v, mask=m)
        return cnt + plsc.all_reduce_population_count(m)[0]

    lcnt = lax.fori_loop(0, BATCH // LANES, prescan, jnp.int32(0))
    nlv = (lcnt + LANES - 1) // LANES

    def chunk_col(c):
        cb = jnp.minimum(bs + c * CPB, NBLK - CPB)
        return pl.multiple_of(cb * 128, 128)

    def fire(c, buf):
        pltpu.async_copy(tT_hbm.at[:, pl.ds(chunk_col(c), CHW)],
                         st_v.at[buf], sem_in)

    def drain_in():
        pltpu.make_async_copy(tT_hbm.at[:, pl.ds(0, CHW)], st_v.at[0],
                              sem_in).wait()

    fire(jnp.int32(0), 0)

    def chunk_body(c, fired):
        buf = lax.rem(c, 2)
        @pl.when(c + 1 < NCHK)
        def _():
            fire(c + 1, 1 - buf)
        drain_in()
        col0 = chunk_col(c)
        lo = col0 >> 7
        hi = lo + CPB

        def list_vreg(j, fired):
            vi = il_v[pl.ds(j * LANES, LANES)]
            vb = bl_v[pl.ds(j * LANES, LANES)]
            valid = (lane + j * LANES) < lcnt
            m0 = ((vi >> 7) >= lo) & ((vi >> 7) < hi) & valid

            def member(k, carry):
                m, fired = carry
                j1 = plsc.all_reduce_ffs(m != 0)
                idx_s = _dyn_gather(vi, j1)[0]
                b_s = _dyn_gather(vb, j1)[0]
                col = jnp.full((LANES,), idx_s - col0, jnp.int32)
                slot = lax.rem(fired, RING)
                @pl.when(fired >= RING)
                def _():
                    pltpu.make_async_copy(
                        rb_v.at[pl.ds(0, DIM)],
                        rows_hbm.at[pl.ds(0, DIM)], sem_out).wait()
                for k4 in range(DIM // LANES):
                    rows = lane + (k4 * LANES)
                    rv = plsc.load_gather(st_v.at[buf], [rows, col])
                    rb_v[pl.ds(slot * DIM + k4 * LANES, LANES)] = rv
                pltpu.async_copy(rb_v.at[pl.ds(slot * DIM, DIM)],
                                 rows_hbm.at[pl.ds(b_s * DIM, DIM)], sem_out)
                m = m & (lane != j1[0]).astype(jnp.int32)
                return m, fired + 1

            n0 = plsc.all_reduce_population_count(m0)[0]
            _, fired = lax.fori_loop(0, n0, member,
                                     (m0.astype(jnp.int32), fired))
            return fired

        return lax.fori_loop(0, nlv, list_vreg, fired)

    fired = lax.fori_loop(0, NCHK, chunk_body, jnp.int32(0))

    def drain_out(i, carry):
        @pl.when(i < jnp.minimum(fired, RING))
        def _():
            pltpu.make_async_copy(rb_v.at[pl.ds(0, DIM)],
                                  rows_hbm.at[pl.ds(0, DIM)], sem_out).wait()
        return carry

    lax.fori_loop(0, RING, drain_out, 0)


@functools.partial(
    pl.kernel,
    mesh=_mesh,
    out_type=jax.ShapeDtypeStruct((BATCH,), jnp.float32),
    scratch_types=[
        pltpu.VMEM((BPW * DIM,), jnp.float32),
        pltpu.VMEM((BPW * DIM,), jnp.float32),
        pltpu.VMEM((BPW,), jnp.float32),
        pltpu.VMEM((LANES * TPAD,), jnp.float32),
    ],
    compiler_params=_params,
)
def _dot_phase(crows_hbm, xrows_hbm, out_hbm, cr_v, xr_v, o_v, tb_v):
    wid = lax.axis_index("s") * NC + lax.axis_index("c")
    base = wid * BPW
    pltpu.sync_copy(crows_hbm.at[pl.ds(base * DIM, BPW * DIM)], cr_v)
    pltpu.sync_copy(xrows_hbm.at[pl.ds(base * DIM, BPW * DIM)], xr_v)
    lane = lax.iota(jnp.int32, LANES)

    def group(g, carry):
        w0 = g * LANES * DIM
        for j in range(LANES):
            w = w0 + j * DIM
            acc = cr_v[pl.ds(w, LANES)] * xr_v[pl.ds(w, LANES)]
            for k in range(1, DIM // LANES):
                acc = acc + (cr_v[pl.ds(w + k * LANES, LANES)]
                             * xr_v[pl.ds(w + k * LANES, LANES)])
            tb_v[pl.ds(j * TPAD, LANES)] = acc
        tot = plsc.load_gather(tb_v, [lane * TPAD])
        for i in range(1, LANES):
            tot = tot + plsc.load_gather(tb_v, [lane * TPAD + i])
        o_v[pl.ds(g * LANES, LANES)] = tot
        return carry

    lax.fori_loop(0, NGRP, group, 0)
    pltpu.sync_copy(o_v, out_hbm.at[pl.ds(base, BPW)])


def kernel(center_word, context_word, center_table, context_table):
    cw = center_word.astype(jnp.int32)
    xw = context_word.astype(jnp.int32)
    crows = _scan_gather(cw, jnp.transpose(center_table))
    xrows = _scan_gather(xw, jnp.transpose(context_table))
    return _dot_phase(crows, xrows)
